# baseline (speedup 1.0000x reference)
import jax
import jax.numpy as jnp
from jax.experimental import pallas as pl
from jax.experimental.pallas import tpu as pltpu

_N = 1024 * 1024
_U = _N // 32
_CHUNKS = [6 * _U, 6 * _U, 5 * _U, 4 * _U, 3 * _U, 3 * _U, 2 * _U, _U, _U, _U]
_NCH = len(_CHUNKS)
_OFFS = [sum(_CHUNKS[:g]) for g in range(_NCH)]


def _body(w_hbm, n_hbm, o_hbm, *scr):
    wv = scr[0:_NCH]
    nv = scr[_NCH:2 * _NCH]
    ov = scr[2 * _NCH:3 * _NCH]
    sw = scr[3 * _NCH:4 * _NCH]
    sn = scr[4 * _NCH:5 * _NCH]
    so = scr[5 * _NCH:6 * _NCH]

    h_in = []
    for g in range(_NCH):
        hw = pltpu.make_async_copy(
            w_hbm.at[pl.ds(_OFFS[g], _CHUNKS[g])], wv[g], sw[g])
        hn = pltpu.make_async_copy(
            n_hbm.at[pl.ds(_OFFS[g], _CHUNKS[g])], nv[g], sn[g])
        hw.start()
        hn.start()
        h_in.append((hw, hn))

    h_out = []
    for g in range(_NCH):
        hw, hn = h_in[g]
        hw.wait()
        hn.wait()
        ov[g][...] = jax.nn.sigmoid((wv[g][...] - nv[g][...]) * 10.0)
        ho = pltpu.make_async_copy(
            ov[g], o_hbm.at[pl.ds(_OFFS[g], _CHUNKS[g])], so[g])
        ho.start()
        h_out.append(ho)
    for ho in h_out:
        ho.wait()


def kernel(weights, noises):
    return pl.pallas_call(
        _body,
        out_shape=jax.ShapeDtypeStruct((_N,), jnp.float32),
        in_specs=[
            pl.BlockSpec(memory_space=pl.ANY),
            pl.BlockSpec(memory_space=pl.ANY),
        ],
        out_specs=pl.BlockSpec(memory_space=pl.ANY),
        scratch_shapes=(
            [pltpu.VMEM((c,), jnp.float32) for c in _CHUNKS] * 3
            + [pltpu.SemaphoreType.DMA for _ in range(3 * _NCH)]
        ),
    )(weights, noises)


# speedup vs baseline: 1.0880x; 1.0880x over previous
import jax
import jax.numpy as jnp
from jax.experimental import pallas as pl
from jax.experimental.pallas import tpu as pltpu

_N = 1024 * 1024
_U = _N // 32
_CHUNKS = [12 * _U, 10 * _U, 6 * _U, 2 * _U, _U, _U]
_NCH = len(_CHUNKS)
_OFFS = [sum(_CHUNKS[:g]) for g in range(_NCH)]


def _body(w_hbm, n_hbm, o_hbm, *scr):
    wv = scr[0:_NCH]
    nv = scr[_NCH:2 * _NCH]
    ov = scr[2 * _NCH:3 * _NCH]
    sw = scr[3 * _NCH:4 * _NCH]
    sn = scr[4 * _NCH:5 * _NCH]
    so = scr[5 * _NCH:6 * _NCH]

    h_in = []
    for g in range(_NCH):
        hw = pltpu.make_async_copy(
            w_hbm.at[pl.ds(_OFFS[g], _CHUNKS[g])], wv[g], sw[g])
        hn = pltpu.make_async_copy(
            n_hbm.at[pl.ds(_OFFS[g], _CHUNKS[g])], nv[g], sn[g])
        hw.start()
        hn.start()
        h_in.append((hw, hn))

    h_out = []
    for g in range(_NCH):
        hw, hn = h_in[g]
        hw.wait()
        hn.wait()
        ov[g][...] = jax.nn.sigmoid((wv[g][...] - nv[g][...]) * 10.0)
        ho = pltpu.make_async_copy(
            ov[g], o_hbm.at[pl.ds(_OFFS[g], _CHUNKS[g])], so[g])
        ho.start()
        h_out.append(ho)
    for ho in h_out:
        ho.wait()


def kernel(weights, noises):
    return pl.pallas_call(
        _body,
        out_shape=jax.ShapeDtypeStruct((_N,), jnp.float32),
        in_specs=[
            pl.BlockSpec(memory_space=pl.ANY),
            pl.BlockSpec(memory_space=pl.ANY),
        ],
        out_specs=pl.BlockSpec(memory_space=pl.ANY),
        scratch_shapes=(
            [pltpu.VMEM((c,), jnp.float32) for c in _CHUNKS] * 3
            + [pltpu.SemaphoreType.DMA for _ in range(3 * _NCH)]
        ),
    )(weights, noises)
